# Initial kernel scaffold; baseline (speedup 1.0000x reference)
import jax
import jax.numpy as jnp
from jax.experimental import pallas as pl


def kernel(h, adj, edge_index, W_gat, attn_l, attn_r, b_gat, W_gcn, b_gcn, Wp1, bp1, Wp2):
    raise NotImplementedError("write your pallas kernel here")


# speedup vs baseline: 14.2106x; 14.2106x over previous
import functools

import jax
import jax.numpy as jnp
from jax import lax
from jax.experimental import pallas as pl
from jax.experimental.pallas import tpu as pltpu
from jax.experimental.pallas import tpu_sc as plsc

N = 10000
E = 320000
F = 128

_INTERPRET = False


def _elu(x):
    return jnp.where(x > 0, x, jnp.exp(jnp.minimum(x, 0.0)) - 1.0)


def _pre_body(h_ref, wg_ref, wc_ref, al_ref, ar_ref,
              feat_ref, el_ref, er_ref, sup_ref):
    hb = h_ref[...]
    feat = jnp.dot(hb, wg_ref[...], preferred_element_type=jnp.float32)
    feat_ref[...] = feat
    sup_ref[...] = jnp.dot(hb, wc_ref[...], preferred_element_type=jnp.float32)
    el_ref[...] = jnp.sum(feat * al_ref[...], axis=1, keepdims=True)
    er_ref[...] = jnp.sum(feat * ar_ref[...], axis=1, keepdims=True)


def _pre(h, W_gat, W_gcn, attn_l, attn_r):
    BM = 1000
    grid = (N // BM,)
    out_shapes = (
        jax.ShapeDtypeStruct((N, F), jnp.float32),
        jax.ShapeDtypeStruct((N, 1), jnp.float32),
        jax.ShapeDtypeStruct((N, 1), jnp.float32),
        jax.ShapeDtypeStruct((N, F), jnp.float32),
    )
    return pl.pallas_call(
        _pre_body,
        grid=grid,
        in_specs=[
            pl.BlockSpec((BM, F), lambda i: (i, 0)),
            pl.BlockSpec((F, F), lambda i: (0, 0)),
            pl.BlockSpec((F, F), lambda i: (0, 0)),
            pl.BlockSpec((1, F), lambda i: (0, 0)),
            pl.BlockSpec((1, F), lambda i: (0, 0)),
        ],
        out_specs=(
            pl.BlockSpec((BM, F), lambda i: (i, 0)),
            pl.BlockSpec((BM, 1), lambda i: (i, 0)),
            pl.BlockSpec((BM, 1), lambda i: (i, 0)),
            pl.BlockSpec((BM, F), lambda i: (i, 0)),
        ),
        out_shape=out_shapes,
        interpret=_INTERPRET,
    )(h, W_gat, W_gcn, attn_l.reshape(1, F), attn_r.reshape(1, F))


def _gcn_body(adj_ref, sup_ref, b_ref, out_ref, s1_ref):
    m = pl.program_id(0)
    emb1 = _elu(jnp.dot(adj_ref[...], sup_ref[...],
                        preferred_element_type=jnp.float32) + b_ref[...])
    out_ref[...] = emb1

    @pl.when(m == 0)
    def _init():
        s1_ref[...] = jnp.zeros_like(s1_ref)

    s1_ref[...] += jnp.sum(emb1, axis=0, keepdims=True)


def _gcn(adj, support, b_gcn):
    BM = 400
    grid = (N // BM,)
    return pl.pallas_call(
        _gcn_body,
        grid=grid,
        in_specs=[
            pl.BlockSpec((BM, N), lambda m: (m, 0)),
            pl.BlockSpec((N, F), lambda m: (0, 0)),
            pl.BlockSpec((1, F), lambda m: (0, 0)),
        ],
        out_specs=(
            pl.BlockSpec((BM, F), lambda m: (m, 0)),
            pl.BlockSpec((1, F), lambda m: (0, 0)),
        ),
        out_shape=(
            jax.ShapeDtypeStruct((N, F), jnp.float32),
            jax.ShapeDtypeStruct((1, F), jnp.float32),
        ),
        interpret=_INTERPRET,
    )(adj, support, b_gcn.reshape(1, F))


NTILES = 32
EPT = E // NTILES
CH = 128
NCH = (EPT + CH - 1) // CH
EPAD = NCH * CH
PW = 144
RPT = 624


def _edge_body(src_hbm, dst_hbm, el_hbm, er_hbm, feat_hbm, agg_hbm, den_hbm,
               el_v, er_v, src_row, dst_row, rows_v, den_v, agg_sh, gsem):
    sc = lax.axis_index("c")
    sid = lax.axis_index("s")
    wid = sc * 16 + sid

    zero16 = jnp.zeros((16,), jnp.float32)

    def _zrow(r, _):
        for g in range(F // 16):
            rows_v[r, pl.ds(g * 16, 16)] = zero16
        return 0
    lax.fori_loop(0, CH, _zrow, 0)
    for t in range(4):
        pltpu.sync_copy(rows_v, agg_sh.at[pl.ds(sid * RPT + t * CH, CH)])
    pltpu.sync_copy(rows_v.at[pl.ds(0, RPT - 4 * CH)],
                    agg_sh.at[pl.ds(sid * RPT + 4 * CH, RPT - 4 * CH)])

    @pl.when(sid == 15)
    def _ztail():
        pltpu.sync_copy(rows_v.at[pl.ds(0, N - 16 * RPT)],
                        agg_sh.at[pl.ds(16 * RPT, N - 16 * RPT)])

    def _zden(k, _):
        den_v[pl.ds(k * 16, 16)] = zero16
        return 0
    lax.fori_loop(0, N // 16, _zden, 0)

    pltpu.sync_copy(el_hbm, el_v)
    pltpu.sync_copy(er_hbm, er_v)

    plsc.subcore_barrier()

    iota16 = lax.iota(jnp.int32, 16)

    def _chunk(j, _):
        pltpu.sync_copy(src_hbm.at[wid, j], src_row)
        pltpu.sync_copy(dst_hbm.at[wid, j], dst_row)
        pltpu.async_copy(feat_hbm.at[src_row], rows_v, gsem).wait()

        def _q(q, _):
            s16 = src_row[pl.ds(q * 16, 16)]
            d16 = dst_row[pl.ds(q * 16, 16)]
            x = plsc.load_gather(el_v, [s16]) + plsc.load_gather(er_v, [d16])
            x = jnp.maximum(x, 0.2 * x)
            ee16 = jnp.exp(x)
            ids = iota16 + (j * CH + q * 16)
            ee16 = jnp.where(ids < EPT, ee16, 0.0)
            plsc.addupdate_scatter(den_v, [d16], ee16)
            for r in range(16):
                s = ee16[r]
                row = q * 16 + r
                for g in range(F // 16):
                    rows_v[row, pl.ds(g * 16, 16)] = (
                        rows_v[row, pl.ds(g * 16, 16)] * s)
            return 0
        lax.fori_loop(0, CH // 16, _q, 0)

        pltpu.sync_copy(rows_v, agg_sh.at[dst_row], add=True)
        return 0
    lax.fori_loop(0, NCH, _chunk, 0)

    plsc.subcore_barrier()
    pltpu.sync_copy(agg_sh.at[pl.ds(sid * RPT, RPT)],
                    agg_hbm.at[sc, pl.ds(sid * RPT, RPT)])

    @pl.when(sid == 15)
    def _wtail():
        pltpu.sync_copy(agg_sh.at[pl.ds(16 * RPT, N - 16 * RPT)],
                        agg_hbm.at[sc, pl.ds(16 * RPT, N - 16 * RPT)])

    pltpu.sync_copy(den_v, den_hbm.at[wid])


def _edge_agg(edge_index, el, er, feat):
    src = edge_index[0].astype(jnp.int32)
    dst = edge_index[1].astype(jnp.int32)
    pad = EPAD - EPT
    src3 = jnp.pad(src.reshape(NTILES, EPT), ((0, 0), (0, pad))).reshape(
        NTILES, NCH, CH)
    dst3 = jnp.pad(dst.reshape(NTILES, EPT), ((0, 0), (0, pad))).reshape(
        NTILES, NCH, CH)
    f = pl.kernel(
        _edge_body,
        out_type=(
            jax.ShapeDtypeStruct((2, N, F), jnp.float32),
            jax.ShapeDtypeStruct((NTILES, N), jnp.float32),
        ),
        mesh=plsc.VectorSubcoreMesh(core_axis_name="c", subcore_axis_name="s"),
        compiler_params=pltpu.CompilerParams(needs_layout_passes=False,
                                             use_tc_tiling_on_sc=False),
        scratch_types=[
            pltpu.VMEM((N,), jnp.float32),
            pltpu.VMEM((N,), jnp.float32),
            pltpu.VMEM((CH,), jnp.int32),
            pltpu.VMEM((CH,), jnp.int32),
            pltpu.VMEM((CH, F), jnp.float32),
            pltpu.VMEM((N,), jnp.float32),
            pltpu.VMEM_SHARED((N, F), jnp.float32),
            pltpu.SemaphoreType.DMA,
        ],
    )
    return f(src3, dst3, el, er, feat)


def _combine_body(aggp_ref, denp_ref, emb1_ref, s1_ref, bg_ref, wp1_ref,
                  bp1_ref, wp2_ref, emb0_ref, g_ref, w0_ref, w1_ref):
    i = pl.program_id(0)
    agg = aggp_ref[0] + aggp_ref[1]
    den = jnp.sum(denp_ref[...], axis=0)
    emb0 = _elu(agg / jnp.maximum(den, 1e-9) + bg_ref[...])
    emb0_ref[...] = emb0
    e2c = emb1_ref[...] - s1_ref[...] * (1.0 / N)

    @pl.when(i == 0)
    def _init():
        g_ref[...] = jnp.zeros_like(g_ref)
        w0_ref[...] = jnp.zeros_like(w0_ref)
        w1_ref[...] = jnp.zeros_like(w1_ref)

    g_ref[...] += lax.dot_general(emb0, e2c, (((0,), (0,)), ((), ())),
                                  preferred_element_type=jnp.float32)
    t0 = jnp.tanh(jnp.dot(emb0, wp1_ref[...],
                          preferred_element_type=jnp.float32) + bp1_ref[...])
    t1 = jnp.tanh(jnp.dot(emb1_ref[...], wp1_ref[...],
                          preferred_element_type=jnp.float32) + bp1_ref[...])
    w0_ref[...] += jnp.sum(jnp.dot(t0, wp2_ref[...],
                                   preferred_element_type=jnp.float32),
                           axis=0, keepdims=True)
    w1_ref[...] += jnp.sum(jnp.dot(t1, wp2_ref[...],
                                   preferred_element_type=jnp.float32),
                           axis=0, keepdims=True)


def _combine(aggp, denp, emb1, s1, b_gat, Wp1, bp1, Wp2):
    BM = 1000
    grid = (N // BM,)
    out_shapes = (
        jax.ShapeDtypeStruct((N, F), jnp.float32),
        jax.ShapeDtypeStruct((F, F), jnp.float32),
        jax.ShapeDtypeStruct((1, 1), jnp.float32),
        jax.ShapeDtypeStruct((1, 1), jnp.float32),
    )
    return pl.pallas_call(
        _combine_body,
        grid=grid,
        in_specs=[
            pl.BlockSpec((2, BM, F), lambda i: (0, i, 0)),
            pl.BlockSpec((NTILES, BM, 1), lambda i: (0, i, 0)),
            pl.BlockSpec((BM, F), lambda i: (i, 0)),
            pl.BlockSpec((1, F), lambda i: (0, 0)),
            pl.BlockSpec((1, F), lambda i: (0, 0)),
            pl.BlockSpec((F, F), lambda i: (0, 0)),
            pl.BlockSpec((1, F), lambda i: (0, 0)),
            pl.BlockSpec((F, 1), lambda i: (0, 0)),
        ],
        out_specs=(
            pl.BlockSpec((BM, F), lambda i: (i, 0)),
            pl.BlockSpec((F, F), lambda i: (0, 0)),
            pl.BlockSpec((1, 1), lambda i: (0, 0)),
            pl.BlockSpec((1, 1), lambda i: (0, 0)),
        ),
        out_shape=out_shapes,
        interpret=_INTERPRET,
    )(aggp, denp.reshape(NTILES, N, 1), emb1, s1,
      b_gat.reshape(1, F), Wp1, bp1.reshape(1, F), Wp2)


def _final_body(emb0_ref, emb1_ref, g_ref, w0_ref, w1_ref,
                zout_ref, loss_ref):
    i = pl.program_id(0)
    w0 = w0_ref[0, 0] * (1.0 / N)
    w1 = w1_ref[0, 0] * (1.0 / N)
    m = jnp.maximum(w0, w1)
    e0 = jnp.exp(w0 - m)
    e1 = jnp.exp(w1 - m)
    b0 = e0 / (e0 + e1)
    b1 = e1 / (e0 + e1)
    zout_ref[...] = b0 * emb0_ref[...] + b1 * emb1_ref[...]

    @pl.when(i == 0)
    def _loss():
        B = g_ref[...]
        loss_ref[...] = jnp.sum(B * B).reshape(1, 1)


def _final(emb0, emb1, G, w0, w1):
    BM = 1000
    grid = (N // BM,)
    out_shapes = (
        jax.ShapeDtypeStruct((N, F), jnp.float32),
        jax.ShapeDtypeStruct((1, 1), jnp.float32),
    )
    return pl.pallas_call(
        _final_body,
        grid=grid,
        in_specs=[
            pl.BlockSpec((BM, F), lambda i: (i, 0)),
            pl.BlockSpec((BM, F), lambda i: (i, 0)),
            pl.BlockSpec((F, F), lambda i: (0, 0)),
            pl.BlockSpec((1, 1), lambda i: (0, 0)),
            pl.BlockSpec((1, 1), lambda i: (0, 0)),
        ],
        out_specs=(
            pl.BlockSpec((BM, F), lambda i: (i, 0)),
            pl.BlockSpec((1, 1), lambda i: (0, 0)),
        ),
        out_shape=out_shapes,
        interpret=_INTERPRET,
    )(emb0, emb1, G, w0, w1)


def kernel(h, adj, edge_index, W_gat, attn_l, attn_r, b_gat,
           W_gcn, b_gcn, Wp1, bp1, Wp2):
    feat, el, er, support = _pre(h, W_gat, W_gcn, attn_l, attn_r)
    emb1, s1 = _gcn(adj, support, b_gcn)
    aggp, denp = _edge_agg(edge_index, el[:, 0], er[:, 0], feat)
    emb0, G, w0, w1 = _combine(aggp, denp, emb1, s1, b_gat, Wp1, bp1, Wp2)
    zout, loss = _final(emb0, emb1, G, w0, w1)
    return zout, loss[0, 0]
